# Initial kernel scaffold; baseline (speedup 1.0000x reference)
import jax
import jax.numpy as jnp
from jax.experimental import pallas as pl


def kernel(risk, time, event):
    raise NotImplementedError("write your pallas kernel here")


# speedup vs baseline: 2.4604x; 2.4604x over previous
import jax
import jax.numpy as jnp
from jax import lax
from jax.experimental import pallas as pl

R, C = 256, 256
N = R * C


def _partner(x, d, row, col):
    if d < C:
        low = (col & d) == 0
        return jnp.where(low, jnp.roll(x, -d, axis=1), jnp.roll(x, d, axis=1))
    d2 = d // C
    low = (row & d2) == 0
    return jnp.where(low, jnp.roll(x, -d2, axis=0), jnp.roll(x, d2, axis=0))


def _loss_body(r_ref, t_ref, e_ref, o_ref):
    r = r_ref[...]
    t = t_ref[...]
    ev = e_ref[...]

    row = lax.broadcasted_iota(jnp.int32, (R, C), 0)
    col = lax.broadcasted_iota(jnp.int32, (R, C), 1)

    k1 = lax.bitcast_convert_type(t, jnp.int32)
    k2 = (row * C + col) * 2 + ev.astype(jnp.int32)

    kk = 2
    while kk <= N:
        d = kk // 2
        while d >= 1:
            if d < C:
                is_low = (col & d) == 0
            else:
                is_low = (row & (d // C)) == 0
            if kk < C:
                asc = (col & kk) == 0
            elif kk < N:
                asc = (row & (kk // C)) == 0
            else:
                asc = jnp.full((R, C), True)
            k1p = _partner(k1, d, row, col)
            k2p = _partner(k2, d, row, col)
            rp = _partner(r, d, row, col)
            pred_b = (k1p > k1) | ((k1p == k1) & (k2p < k2))
            keep = (is_low == asc) ^ pred_b
            k1 = jnp.where(keep, k1, k1p)
            k2 = jnp.where(keep, k2, k2p)
            r = jnp.where(keep, r, rp)
            d //= 2
        kk *= 2

    ev_s = (k2 & 1).astype(jnp.float32)
    m = jnp.max(r)
    ex = jnp.exp(r - m)
    upper = (row <= col).astype(jnp.float32)
    cum = lax.dot_general(ex, upper, (((1,), (0,)), ((), ())),
                          precision=lax.Precision.HIGHEST,
                          preferred_element_type=jnp.float32)
    row_tot = cum[:, C - 1:C]
    strict_lower = (col < row).astype(jnp.float32)
    base = lax.dot_general(strict_lower, row_tot, (((1,), (0,)), ((), ())),
                           precision=lax.Precision.HIGHEST,
                           preferred_element_type=jnp.float32)
    den = jnp.maximum(cum + base, 1e-37)
    log_den = m + jnp.log(den)

    s_evrisk = jnp.sum(ev_s * r)
    s_ev = jnp.sum(ev_s)
    s_logden = jnp.sum(ev_s * log_den)
    loss = -(s_evrisk - s_logden) / (s_ev + 1e-8)
    o_ref[...] = jnp.broadcast_to(loss, (8, 128))


def kernel(risk, time, event, interpret=False):
    out = pl.pallas_call(
        _loss_body,
        out_shape=jax.ShapeDtypeStruct((8, 128), jnp.float32),
        interpret=interpret,
    )(risk.reshape(R, C), time.reshape(R, C), event.reshape(R, C))
    return out[0, 0]


# speedup vs baseline: 3.0929x; 1.2571x over previous
import jax
import jax.numpy as jnp
from jax import lax
from jax.experimental import pallas as pl

R, C = 256, 256
N = R * C


def _partner(x, d, row, col):
    if d < C:
        low = (col & d) == 0
        return jnp.where(low, jnp.roll(x, -d, axis=1), jnp.roll(x, d, axis=1))
    d2 = d // C
    low = (row & d2) == 0
    return jnp.where(low, jnp.roll(x, -d2, axis=0), jnp.roll(x, d2, axis=0))


def _loss_body(r_ref, t_ref, e_ref, o_ref):
    r = r_ref[...]
    t = t_ref[...]
    ev = e_ref[...]

    row = lax.broadcasted_iota(jnp.int32, (R, C), 0)
    col = lax.broadcasted_iota(jnp.int32, (R, C), 1)

    k = (lax.bitcast_convert_type(t, jnp.int32) << 1) | ev.astype(jnp.int32)

    kk = 2
    while kk <= N:
        d = kk // 2
        while d >= 1:
            if d < C:
                is_low = (col & d) == 0
            else:
                is_low = (row & (d // C)) == 0
            if kk < C:
                asc = (col & kk) == 0
            elif kk < N:
                asc = (row & (kk // C)) == 0
            else:
                asc = jnp.full((R, C), True)
            kp = _partner(k, d, row, col)
            rp = _partner(r, d, row, col)
            te = asc == is_low
            swap = (te & (kp > k)) | (~te & (kp < k))
            k = jnp.where(swap, kp, k)
            r = jnp.where(swap, rp, r)
            d //= 2
        kk *= 2

    ev_s = (k & 1).astype(jnp.float32)
    m = jnp.max(r)
    ex = jnp.exp(r - m)
    upper = (row <= col).astype(jnp.float32)
    cum = lax.dot_general(ex, upper, (((1,), (0,)), ((), ())),
                          precision=lax.Precision.HIGHEST,
                          preferred_element_type=jnp.float32)
    row_tot = cum[:, C - 1:C]
    strict_lower = (col < row).astype(jnp.float32)
    base = lax.dot_general(strict_lower, row_tot, (((1,), (0,)), ((), ())),
                           precision=lax.Precision.HIGHEST,
                           preferred_element_type=jnp.float32)
    den = jnp.maximum(cum + base, 1e-37)
    log_den = m + jnp.log(den)

    s_evrisk = jnp.sum(ev_s * r)
    s_ev = jnp.sum(ev_s)
    s_logden = jnp.sum(ev_s * log_den)
    loss = -(s_evrisk - s_logden) / (s_ev + 1e-8)
    o_ref[...] = jnp.broadcast_to(loss, (8, 128))


def kernel(risk, time, event, interpret=False):
    out = pl.pallas_call(
        _loss_body,
        out_shape=jax.ShapeDtypeStruct((8, 128), jnp.float32),
        interpret=interpret,
    )(risk.reshape(R, C), time.reshape(R, C), event.reshape(R, C))
    return out[0, 0]
